# baseline (speedup 1.0000x reference)
import functools

import jax
import jax.numpy as jnp
from jax import lax
from jax.experimental import pallas as pl
from jax.experimental.pallas import tpu as pltpu
from jax.experimental.pallas import tpu_sc as plsc

_N_BLOCKS = 512
_KH = _KW = 16
_H = _W = 512
_GH = _H // _KH
_GW = _W // _KW
_K = _N_BLOCKS
_B = 32
_NC = 2
_L = 16


def _body(fidx_hbm, seq_hbm, blocks_hbm, out_hbm,
          fidx_v, logits_a, logits_b, idx_a, idx_b, band_a, band_b,
          sem_la, sem_lb, sem_ga, sem_gb, sem_sa, sem_sb):
    b = lax.axis_index("s") * _NC + lax.axis_index("c")
    iota = lax.iota(jnp.int32, _L)
    t_lo = iota
    t_hi = iota + _L

    pltpu.sync_copy(fidx_hbm, fidx_v.at[pl.ds(0, _B)])
    fvec = plsc.load_gather(fidx_v, [jnp.full((_L,), b, jnp.int32)])

    fidx_v[pl.ds(_B, _L)] = fvec
    fscal = fidx_v[pl.ds(_B, _L)][0]

    def start_load(gy, logits_v, sem):
        row0 = jnp.minimum(gy, _GH - 1) * _GW
        pltpu.async_copy(seq_hbm.at[fscal, pl.ds(row0, _GW)], logits_v, sem)

    def wait_load(logits_v, sem):
        pltpu.make_async_copy(seq_hbm.at[0, pl.ds(0, _GW)],
                              logits_v, sem).wait()

    def amax(logits_v):
        def outer(q, st):
            m0, mi0, m1, mi1 = st
            for j in range(8):
                ev = jnp.full((_L,), q * 8 + j, jnp.int32) ^ iota
                v0 = plsc.load_gather(logits_v, [t_lo, ev])
                v1 = plsc.load_gather(logits_v, [t_hi, ev])

                def upd(v, m, mi):
                    tie = jnp.where(v == m, jnp.minimum(mi, ev), mi)
                    return jnp.maximum(v, m), jnp.where(v > m, ev, tie)

                m0, mi0 = upd(v0, m0, mi0)
                m1, mi1 = upd(v1, m1, mi1)
            return (m0, mi0, m1, mi1)

        ninf = jnp.full((_L,), -jnp.inf, jnp.float32)
        zero = jnp.zeros((_L,), jnp.int32)
        _, mi0, _, mi1 = lax.fori_loop(
            0, _K // 8, outer, (ninf, zero, ninf, zero))
        return mi0, mi1

    def fill_idx(idx2_v, mi0, mi1):
        p0 = mi0 * _KH
        p1 = mi1 * _KH
        for r in range(_KH):
            row, off = r // 4, (r % 4) * (2 * _L)
            idx2_v[row, pl.ds(off, _L)] = p0 + r
            idx2_v[row, pl.ds(off + _L, _L)] = p1 + r

    def start_gathers(idx2_v, band_v, sem):
        for j in range(4):
            pltpu.async_copy(blocks_hbm.at[idx2_v.at[j]],
                             band_v.at[pl.ds(j * 128, 128)], sem)

    def wait_gathers(idx2_v, band_v, sem):
        for j in range(4):
            pltpu.make_async_copy(blocks_hbm.at[idx2_v.at[j]],
                                  band_v.at[pl.ds(j * 128, 128)], sem).wait()

    bufs = ((logits_a, idx_a, band_a, sem_la, sem_ga, sem_sa),
            (logits_b, idx_b, band_b, sem_lb, sem_gb, sem_sb))

    def chunk(m, gy, this, prev, first_reuse, prev_exists):
        logits_v, idx2_v, band_v, sem_l, sem_g, sem_s = this
        _, idx2_p, band_p, _, sem_gp, sem_sp = prev

        wait_load(logits_v, sem_l)
        mi0, mi1 = amax(logits_v)
        start_load(gy + 2, logits_v, sem_l)

        @pl.when(first_reuse)
        def _():
            pltpu.make_async_copy(band_v, out_hbm.at[0], sem_s).wait()

        fill_idx(idx2_v, mi0, mi1)
        start_gathers(idx2_v, band_v, sem_g)

        @pl.when(prev_exists)
        def _():
            wait_gathers(idx2_p, band_p, sem_gp)
            pltpu.async_copy(band_p, out_hbm.at[b * _GH + gy - 1], sem_sp)

    start_load(jnp.int32(0), logits_a, sem_la)
    start_load(jnp.int32(1), logits_b, sem_lb)

    def pair(m, carry):
        chunk(m, 2 * m, bufs[0], bufs[1], m >= 1, m >= 1)
        chunk(m, 2 * m + 1, bufs[1], bufs[0], m >= 1, m >= 0)
        return carry

    lax.fori_loop(0, _GH // 2, pair, jnp.int32(0))

    wait_gathers(idx_b, band_b, sem_gb)
    pltpu.async_copy(band_b, out_hbm.at[b * _GH + _GH - 1], sem_sb)
    pltpu.make_async_copy(band_a, out_hbm.at[0], sem_sa).wait()
    pltpu.make_async_copy(band_b, out_hbm.at[0], sem_sb).wait()
    wait_load(logits_a, sem_la)
    wait_load(logits_b, sem_lb)


@jax.jit
def _sc_reconstruct(frame_idxs, seq_rows, blocks16):
    run = pl.kernel(
        _body,
        out_type=jax.ShapeDtypeStruct((_B * _GH, _W, _KW), jnp.float32),
        mesh=plsc.VectorSubcoreMesh(core_axis_name="c", subcore_axis_name="s"),
        compiler_params=pltpu.CompilerParams(
            needs_layout_passes=False, use_tc_tiling_on_sc=False),
        scratch_types=[
            pltpu.VMEM((_B + _L,), jnp.int32),
            pltpu.VMEM((_GW, _K), jnp.float32),
            pltpu.VMEM((_GW, _K), jnp.float32),
            pltpu.VMEM((4, 128), jnp.int32),
            pltpu.VMEM((4, 128), jnp.int32),
            pltpu.VMEM((_KH * _GW, _KW), jnp.float32),
            pltpu.VMEM((_KH * _GW, _KW), jnp.float32),
            pltpu.SemaphoreType.DMA,
            pltpu.SemaphoreType.DMA,
            pltpu.SemaphoreType.DMA,
            pltpu.SemaphoreType.DMA,
            pltpu.SemaphoreType.DMA,
            pltpu.SemaphoreType.DMA,
        ],
    )
    return run(frame_idxs, seq_rows, blocks16)


def kernel(frame_idxs, sequence, blocks):
    blocks16 = blocks.reshape(_N_BLOCKS * _KH, _KW)
    out = _sc_reconstruct(frame_idxs, sequence, blocks16)
    return out.reshape(_B, 1, _H, _W)


# speedup vs baseline: 1.9806x; 1.9806x over previous
import functools

import jax
import jax.numpy as jnp
from jax import lax
from jax.experimental import pallas as pl
from jax.experimental.pallas import tpu as pltpu
from jax.experimental.pallas import tpu_sc as plsc

_N_BLOCKS = 512
_KH = _KW = 16
_H = _W = 512
_GH = _H // _KH
_GW = _W // _KW
_K = _N_BLOCKS
_B = 32
_NC = 2
_L = 16


def _body(fidx_hbm, seq_hbm, blocks_hbm, out_hbm,
          fidx_v, logits_a, logits_b, idx_a, idx_b, band_a, band_b,
          sem_la, sem_lb, sem_ga, sem_gb, sem_sa, sem_sb):
    b = lax.axis_index("s") * _NC + lax.axis_index("c")
    iota = lax.iota(jnp.int32, _L)
    t_lo = iota
    t_hi = iota + _L

    pltpu.sync_copy(fidx_hbm, fidx_v.at[pl.ds(0, _B)])
    fvec = plsc.load_gather(fidx_v, [jnp.full((_L,), b, jnp.int32)])

    fidx_v[pl.ds(_B, _L)] = fvec
    fscal = fidx_v[pl.ds(_B, _L)][0]

    def start_load(gy, logits_v, sem):
        row0 = jnp.minimum(gy, _GH - 1) * 4
        pltpu.async_copy(seq_hbm.at[fscal, pl.ds(row0, 4)], logits_v, sem)

    def wait_load(logits_v, sem):
        pltpu.make_async_copy(seq_hbm.at[0, pl.ds(0, 4)],
                              logits_v, sem).wait()

    i0_lo = jnp.right_shift(t_lo, 3)
    i0_hi = i0_lo + 2
    r128 = jnp.left_shift(t_lo & 7, 7)

    def amax(logits_v):
        def outer(q, st):
            m0, mi0, m1, mi1 = st
            for j in range(8):
                ev = jnp.full((_L,), q * 8 + j, jnp.int32) ^ iota
                col = r128 + (jnp.left_shift(ev & 0x180, 3) | (ev & 127))
                v0 = plsc.load_gather(logits_v, [i0_lo, col])
                v1 = plsc.load_gather(logits_v, [i0_hi, col])

                def upd(v, m, mi):
                    tie = jnp.where(v == m, jnp.minimum(mi, ev), mi)
                    return jnp.maximum(v, m), jnp.where(v > m, ev, tie)

                m0, mi0 = upd(v0, m0, mi0)
                m1, mi1 = upd(v1, m1, mi1)
            return (m0, mi0, m1, mi1)

        ninf = jnp.full((_L,), -jnp.inf, jnp.float32)
        zero = jnp.zeros((_L,), jnp.int32)
        _, mi0, _, mi1 = lax.fori_loop(
            0, _K // 8, outer, (ninf, zero, ninf, zero))
        return mi0, mi1

    def fill_idx(idx2_v, mi0, mi1):
        p0 = mi0 * _KH
        p1 = mi1 * _KH
        for r in range(_KH):
            row, off = r // 4, (r % 4) * (2 * _L)
            idx2_v[row, pl.ds(off, _L)] = p0 + r
            idx2_v[row, pl.ds(off + _L, _L)] = p1 + r

    def start_gathers(idx2_v, band_v, sem):
        for j in range(4):
            pltpu.async_copy(blocks_hbm.at[idx2_v.at[j]],
                             band_v.at[pl.ds(j * 128, 128)], sem)

    def wait_gathers(idx2_v, band_v, sem):
        for j in range(4):
            pltpu.make_async_copy(blocks_hbm.at[idx2_v.at[j]],
                                  band_v.at[pl.ds(j * 128, 128)], sem).wait()

    bufs = ((logits_a, idx_a, band_a, sem_la, sem_ga, sem_sa),
            (logits_b, idx_b, band_b, sem_lb, sem_gb, sem_sb))

    def chunk(m, gy, this, prev, first_reuse, prev_exists):
        logits_v, idx2_v, band_v, sem_l, sem_g, sem_s = this
        _, idx2_p, band_p, _, sem_gp, sem_sp = prev

        wait_load(logits_v, sem_l)
        mi0, mi1 = amax(logits_v)
        start_load(gy + 2, logits_v, sem_l)

        @pl.when(first_reuse)
        def _():
            pltpu.make_async_copy(band_v, out_hbm.at[0], sem_s).wait()

        fill_idx(idx2_v, mi0, mi1)
        start_gathers(idx2_v, band_v, sem_g)

        @pl.when(prev_exists)
        def _():
            wait_gathers(idx2_p, band_p, sem_gp)
            pltpu.async_copy(band_p, out_hbm.at[b * _GH + gy - 1], sem_sp)

    start_load(jnp.int32(0), logits_a, sem_la)
    start_load(jnp.int32(1), logits_b, sem_lb)

    def pair(m, carry):
        chunk(m, 2 * m, bufs[0], bufs[1], m >= 1, m >= 1)
        chunk(m, 2 * m + 1, bufs[1], bufs[0], m >= 1, m >= 0)
        return carry

    lax.fori_loop(0, _GH // 2, pair, jnp.int32(0))

    wait_gathers(idx_b, band_b, sem_gb)
    pltpu.async_copy(band_b, out_hbm.at[b * _GH + _GH - 1], sem_sb)
    pltpu.make_async_copy(band_a, out_hbm.at[0], sem_sa).wait()
    pltpu.make_async_copy(band_b, out_hbm.at[0], sem_sb).wait()
    wait_load(logits_a, sem_la)
    wait_load(logits_b, sem_lb)


@jax.jit
def _sc_reconstruct(frame_idxs, seq_rows, blocks16):
    run = pl.kernel(
        _body,
        out_type=jax.ShapeDtypeStruct((_B * _GH, _W, _KW), jnp.float32),
        mesh=plsc.VectorSubcoreMesh(core_axis_name="c", subcore_axis_name="s"),
        compiler_params=pltpu.CompilerParams(
            needs_layout_passes=False, use_tc_tiling_on_sc=False),
        scratch_types=[
            pltpu.VMEM((_B + _L,), jnp.int32),
            pltpu.VMEM((4, 4096), jnp.float32),
            pltpu.VMEM((4, 4096), jnp.float32),
            pltpu.VMEM((4, 128), jnp.int32),
            pltpu.VMEM((4, 128), jnp.int32),
            pltpu.VMEM((_KH * _GW, _KW), jnp.float32),
            pltpu.VMEM((_KH * _GW, _KW), jnp.float32),
            pltpu.SemaphoreType.DMA,
            pltpu.SemaphoreType.DMA,
            pltpu.SemaphoreType.DMA,
            pltpu.SemaphoreType.DMA,
            pltpu.SemaphoreType.DMA,
            pltpu.SemaphoreType.DMA,
        ],
    )
    return run(frame_idxs, seq_rows, blocks16)


def kernel(frame_idxs, sequence, blocks):
    n_frames = sequence.shape[0]
    seq_t = jnp.transpose(
        sequence.reshape(n_frames, 128, 8, 4, 128), (0, 1, 3, 2, 4))
    seq_t = seq_t.reshape(n_frames, 128, 4096)
    blocks16 = blocks.reshape(_N_BLOCKS * _KH, _KW)
    out = _sc_reconstruct(frame_idxs, seq_t, blocks16)
    return out.reshape(_B, 1, _H, _W)
